# Initial kernel scaffold; baseline (speedup 1.0000x reference)
import functools

import jax
import jax.numpy as jnp
from jax import lax
from jax.experimental import pallas as pl
from jax.experimental.pallas import tpu as pltpu
from jax.experimental.pallas import tpu_sc as plsc

DIM = 64
B_ROWS = 16384
B_COLS = 50
B_TOTAL = B_ROWS * B_COLS

_info = plsc.get_sparse_core_info()
NC = _info.num_cores
NS = _info.num_subcores
NW = NC * NS
B_PER_W = B_TOTAL // NW

IB = 128
K = 4
CHUNK = K * IB
NCHUNK = B_PER_W // CHUNK
NBUF = 2
NGROUP = NCHUNK // NBUF


@functools.partial(
    pl.kernel,
    mesh=plsc.VectorSubcoreMesh(core_axis_name="c", subcore_axis_name="s"),
    out_type=jax.ShapeDtypeStruct((B_TOTAL, DIM), jnp.float32),
    scratch_types=[
        pltpu.VMEM((NBUF, K, IB), jnp.int32),
        pltpu.VMEM((NBUF, CHUNK, DIM), jnp.float32),
        pltpu.SemaphoreType.DMA,
        pltpu.SemaphoreType.DMA,
    ],
)
def _gather_kernel(idx_hbm, table_hbm, out_hbm, idx_v, rows_v, gat_sem, out_sem):
    wid = lax.axis_index("s") * NC + lax.axis_index("c")
    base = wid * B_PER_W
    base_irow = wid * (B_PER_W // IB)

    def start_gather(c, buf):
        pltpu.sync_copy(idx_hbm.at[pl.ds(base_irow + c * K, K)], idx_v.at[buf])
        for j in range(K):
            pltpu.async_copy(
                table_hbm.at[idx_v.at[buf, j]],
                rows_v.at[buf, pl.ds(j * IB, IB)],
                gat_sem,
            )

    def wait_gather(buf):
        for j in range(K):
            pltpu.make_async_copy(
                table_hbm.at[idx_v.at[buf, j]],
                rows_v.at[buf, pl.ds(j * IB, IB)],
                gat_sem,
            ).wait()

    def wait_writeback(buf):
        pltpu.make_async_copy(
            rows_v.at[buf], out_hbm.at[pl.ds(base, CHUNK)], out_sem
        ).wait()

    start_gather(0, 0)

    def group(g, _):
        c0 = g * NBUF
        for b in range(NBUF):
            c = c0 + b
            nb = (b + 1) % NBUF

            @pl.when(c + 1 < NCHUNK)
            def _():
                @pl.when(c + 1 >= NBUF)
                def _():
                    wait_writeback(nb)
                start_gather(c + 1, nb)

            wait_gather(b)
            pltpu.async_copy(
                rows_v.at[b], out_hbm.at[pl.ds(base + c * CHUNK, CHUNK)],
                out_sem,
            )
        return _

    lax.fori_loop(0, NGROUP, group, None, unroll=False)

    for b in range(NBUF):
        wait_writeback(b)


def kernel(input_, weight):
    idx = input_.reshape(B_TOTAL // IB, IB).astype(jnp.int32)
    out = _gather_kernel(idx, weight)
    return out.reshape(B_ROWS, B_COLS, DIM)


# speedup vs baseline: 1.8547x; 1.8547x over previous
import functools

import jax
import jax.numpy as jnp
from jax import lax
from jax.experimental import pallas as pl
from jax.experimental.pallas import tpu as pltpu
from jax.experimental.pallas import tpu_sc as plsc

DIM = 64
B_ROWS = 16384
B_COLS = 50
B_TOTAL = B_ROWS * B_COLS

_info = plsc.get_sparse_core_info()
NC = _info.num_cores
NS = _info.num_subcores
NW = NC * NS
B_PER_W = B_TOTAL // NW

IB = 128
K = 4
CHUNK = K * IB
NCHUNK = B_PER_W // CHUNK
NBUF = 2
NGROUP = NCHUNK // NBUF


@functools.partial(
    pl.kernel,
    mesh=plsc.VectorSubcoreMesh(core_axis_name="c", subcore_axis_name="s"),
    out_type=jax.ShapeDtypeStruct((B_TOTAL, DIM), jnp.float32),
    scratch_types=[
        pltpu.VMEM((NBUF, K, IB), jnp.int32),
        pltpu.VMEM((NBUF, CHUNK, DIM), jnp.float32),
        pltpu.SemaphoreType.DMA,
        pltpu.SemaphoreType.DMA,
    ],
    compiler_params=pltpu.CompilerParams(use_tc_tiling_on_sc=False),
)
def _gather_kernel(idx_hbm, table_hbm, out_hbm, idx_v, rows_v, gat_sem, out_sem):
    wid = lax.axis_index("s") * NC + lax.axis_index("c")
    base = wid * B_PER_W
    base_irow = wid * (B_PER_W // IB)

    def start_gather(c, buf):
        pltpu.sync_copy(idx_hbm.at[pl.ds(base_irow + c * K, K)], idx_v.at[buf])
        for j in range(K):
            pltpu.async_copy(
                table_hbm.at[idx_v.at[buf, j]],
                rows_v.at[buf, pl.ds(j * IB, IB)],
                gat_sem,
            )

    def wait_gather(buf):
        for j in range(K):
            pltpu.make_async_copy(
                table_hbm.at[idx_v.at[buf, j]],
                rows_v.at[buf, pl.ds(j * IB, IB)],
                gat_sem,
            ).wait()

    def wait_writeback(buf):
        pltpu.make_async_copy(
            rows_v.at[buf], out_hbm.at[pl.ds(base, CHUNK)], out_sem
        ).wait()

    start_gather(0, 0)

    def group(g, _):
        c0 = g * NBUF
        for b in range(NBUF):
            c = c0 + b
            nb = (b + 1) % NBUF

            @pl.when(c + 1 < NCHUNK)
            def _():
                @pl.when(c + 1 >= NBUF)
                def _():
                    wait_writeback(nb)
                start_gather(c + 1, nb)

            wait_gather(b)
            pltpu.async_copy(
                rows_v.at[b], out_hbm.at[pl.ds(base + c * CHUNK, CHUNK)],
                out_sem,
            )
        return _

    lax.fori_loop(0, NGROUP, group, None, unroll=False)

    for b in range(NBUF):
        wait_writeback(b)


def kernel(input_, weight):
    idx = input_.reshape(B_TOTAL // IB, IB).astype(jnp.int32)
    out = _gather_kernel(idx, weight)
    return out.reshape(B_ROWS, B_COLS, DIM)


# speedup vs baseline: 1.8748x; 1.0108x over previous
import functools

import jax
import jax.numpy as jnp
from jax import lax
from jax.experimental import pallas as pl
from jax.experimental.pallas import tpu as pltpu
from jax.experimental.pallas import tpu_sc as plsc

DIM = 64
B_ROWS = 16384
B_COLS = 50
B_TOTAL = B_ROWS * B_COLS

_info = plsc.get_sparse_core_info()
NC = _info.num_cores
NS = _info.num_subcores
NW = NC * NS
B_PER_W = B_TOTAL // NW

IB = 128
IROWS = B_PER_W // IB
K = 4
CHUNK = K * IB
NCHUNK = B_PER_W // CHUNK
NBUF = 2
NGROUP = NCHUNK // NBUF


@functools.partial(
    pl.kernel,
    mesh=plsc.VectorSubcoreMesh(core_axis_name="c", subcore_axis_name="s"),
    out_type=jax.ShapeDtypeStruct((B_TOTAL, DIM), jnp.float32),
    scratch_types=[
        pltpu.VMEM((IROWS, IB), jnp.int32),
        pltpu.VMEM((NBUF, CHUNK, DIM), jnp.float32),
        pltpu.SemaphoreType.DMA,
        pltpu.SemaphoreType.DMA,
    ],
    compiler_params=pltpu.CompilerParams(use_tc_tiling_on_sc=False),
)
def _gather_kernel(idx_hbm, table_hbm, out_hbm, idx_v, rows_v, gat_sem, out_sem):
    wid = lax.axis_index("s") * NC + lax.axis_index("c")
    base = wid * B_PER_W

    pltpu.sync_copy(idx_hbm.at[pl.ds(wid * IROWS, IROWS)], idx_v)

    def start_gather(c, buf):
        for j in range(K):
            pltpu.async_copy(
                table_hbm.at[idx_v.at[c * K + j]],
                rows_v.at[buf, pl.ds(j * IB, IB)],
                gat_sem,
            )

    def wait_gather(c, buf):
        for j in range(K):
            pltpu.make_async_copy(
                table_hbm.at[idx_v.at[c * K + j]],
                rows_v.at[buf, pl.ds(j * IB, IB)],
                gat_sem,
            ).wait()

    def wait_writeback(buf):
        pltpu.make_async_copy(
            rows_v.at[buf], out_hbm.at[pl.ds(base, CHUNK)], out_sem
        ).wait()

    start_gather(0, 0)

    def group(g, _):
        c0 = g * NBUF
        for b in range(NBUF):
            c = c0 + b
            nb = (b + 1) % NBUF

            @pl.when(c + 1 < NCHUNK)
            def _():
                @pl.when(c + 1 >= NBUF)
                def _():
                    wait_writeback(nb)
                start_gather(c + 1, nb)

            wait_gather(c, b)
            pltpu.async_copy(
                rows_v.at[b], out_hbm.at[pl.ds(base + c * CHUNK, CHUNK)],
                out_sem,
            )
        return _

    lax.fori_loop(0, NGROUP, group, None, unroll=False)

    for b in range(NBUF):
        wait_writeback(b)


def kernel(input_, weight):
    idx = input_.reshape(B_TOTAL // IB, IB).astype(jnp.int32)
    out = _gather_kernel(idx, weight)
    return out.reshape(B_ROWS, B_COLS, DIM)
